# baseline (speedup 1.0000x reference)
import jax
import jax.numpy as jnp
from jax.experimental import pallas as pl
from jax.experimental.pallas import tpu as pltpu

_E = 64
_K = 2
_D = 1024
_F = 768
_TM = 128
_G = 95
_P = _G * _TM


_RB = 128
_RG = 16


def _router_body(x_ref, gw_ref, sim_ref, vals_ref, idx_ref, map_ref, meta_ref,
                 xb_ref):
    g = pl.program_id(0)
    xb_ref[...] = x_ref[...].astype(jnp.bfloat16)
    iota_e = jax.lax.broadcasted_iota(jnp.int32, (_RB, _E), 1)
    logits = jax.lax.dot_general(x_ref[...], gw_ref[...],
                                 (((1,), (1,)), ((), ())),
                                 preferred_element_type=jnp.float32)
    m = jnp.max(logits, axis=1, keepdims=True)
    e = jnp.exp(logits - m)
    p = e / jnp.sum(e, axis=1, keepdims=True)
    v0 = jnp.max(p, axis=1)
    i0 = jnp.min(jnp.where(p == v0[:, None], iota_e, _E), axis=1)
    p1 = jnp.where(iota_e == i0[:, None], -1.0, p)
    v1 = jnp.max(p1, axis=1)
    i1 = jnp.min(jnp.where(p1 == v1[:, None], iota_e, _E), axis=1)
    s = jnp.maximum(v0 + v1, 1e-12)
    vals_ref[0, 0, :] = v0 / s
    vals_ref[0, 1, :] = v1 / s
    idx_ref[0, 0, :] = i0
    idx_ref[0, 1, :] = i1

    pm_c = jnp.sum((iota_e == i0[:, None]).astype(jnp.int32), axis=0)
    sm_c = jnp.sum((iota_e == i1[:, None]).astype(jnp.int32), axis=0)

    @pl.when(g == 0)
    def _init():
        map_ref[...] = jnp.zeros_like(map_ref)

    map_ref[1, :] = map_ref[1, :] + pm_c
    map_ref[2, :] = map_ref[2, :] + sm_c

    @pl.when(g == _RG - 1)
    def _finish():
        counts0 = map_ref[1, :]
        counts1 = map_ref[2, :]
        pm = counts0 > 0
        sm = counts1 > 0
        sim = sim_ref[...]
        neg = jnp.float32(-3.0e38)
        cand = jnp.where(pm[None, :], sim, neg)
        best_sim = jnp.max(cand, axis=1)
        iota2 = jax.lax.broadcasted_iota(jnp.int32, (_E, _E), 1)
        best_primary = jnp.min(
            jnp.where(cand == best_sim[:, None], iota2, _E), axis=1)
        any_pm = jnp.sum(pm.astype(jnp.int32)) > 0
        any_sm = jnp.sum(sm.astype(jnp.int32)) > 0
        reroute = sm & (~pm) & any_pm & any_sm & (best_sim >= 0.5)
        ident = jax.lax.broadcasted_iota(jnp.int32, (_E,), 0)
        mapping = jnp.where(reroute, best_primary, ident)
        map_ref[0, :] = mapping

        oh_map = (iota2 == mapping[:, None]).astype(jnp.int32)
        counts_r = counts0 + jnp.sum(oh_map * counts1[:, None], axis=0)
        tile_counts = (counts_r + _TM - 1) // _TM
        r2 = jax.lax.broadcasted_iota(jnp.int32, (_E, _E), 0)
        tile_ends = jnp.sum(
            jnp.where(iota2 <= r2, tile_counts[None, :], 0), axis=1)
        map_ref[3, :] = (tile_ends - tile_counts) * _TM

        nt = jnp.max(tile_ends)
        te_b = jnp.broadcast_to(tile_ends[None, :], (128, _E))
        g_col = jax.lax.broadcasted_iota(jnp.int32, (128, _E), 0)
        gg_col = jnp.minimum(g_col, nt - 1)
        be = jnp.sum((te_b <= gg_col).astype(jnp.int32), axis=1)
        meta_ref[0, :] = be
        meta_ref[1, :] = jnp.full((128,), nt, jnp.int32)


def _router(x, gate_weight, similarity_matrix):
    return pl.pallas_call(
        _router_body,
        grid=(_RG,),
        in_specs=[
            pl.BlockSpec((_RB, _D), lambda g: (g, 0)),
            pl.BlockSpec((_E, _D), lambda g: (0, 0)),
            pl.BlockSpec((_E, _E), lambda g: (0, 0)),
        ],
        out_specs=[
            pl.BlockSpec((1, _K, _RB), lambda g: (g, 0, 0)),
            pl.BlockSpec((1, _K, _RB), lambda g: (g, 0, 0)),
            pl.BlockSpec((8, _E), lambda g: (0, 0)),
            pl.BlockSpec((8, 128), lambda g: (0, 0)),
            pl.BlockSpec((_RB, _D), lambda g: (g, 0)),
        ],
        out_shape=[
            jax.ShapeDtypeStruct((_RG, _K, _RB), jnp.float32),
            jax.ShapeDtypeStruct((_RG, _K, _RB), jnp.int32),
            jax.ShapeDtypeStruct((8, _E), jnp.int32),
            jax.ShapeDtypeStruct((8, 128), jnp.int32),
            jax.ShapeDtypeStruct((_RG * _RB, _D), jnp.bfloat16),
        ],
        compiler_params=pltpu.CompilerParams(
            dimension_semantics=("arbitrary",)),
    )(x, gate_weight, similarity_matrix)


def _dispatch_body(idx_ref, map_ref, dst_ref, run_ref):
    g = pl.program_id(0)
    iota_e = jax.lax.broadcasted_iota(jnp.int32, (_RB, _E), 1)
    i0 = idx_ref[0, 0, :]
    i1 = idx_ref[0, 1, :]
    mapping = map_ref[0, :].astype(jnp.float32)
    pad_off = map_ref[3, :].astype(jnp.float32)

    oh1 = (iota_e == i1[:, None]).astype(jnp.float32)
    e1 = jnp.sum(oh1 * mapping[None, :], axis=1).astype(jnp.int32)
    oh0 = (iota_e == i0[:, None]).astype(jnp.float32)
    oh1m = (iota_e == e1[:, None]).astype(jnp.float32)
    occ = oh0 + oh1m

    r = jax.lax.broadcasted_iota(jnp.int32, (_RB, _RB), 0)
    c = jax.lax.broadcasted_iota(jnp.int32, (_RB, _RB), 1)
    lstrict = (r > c).astype(jnp.float32)
    cnt_lt = jax.lax.dot_general(lstrict, occ, (((1,), (0,)), ((), ())),
                                 preferred_element_type=jnp.float32)

    @pl.when(g == 0)
    def _init():
        run_ref[...] = jnp.zeros_like(run_ref)

    run = run_ref[0, :].astype(jnp.float32)
    base = run[None, :] + cnt_lt + pad_off[None, :]
    dst0 = jnp.sum(oh0 * base, axis=1)
    dst1 = jnp.sum(oh1m * (base + oh0), axis=1)
    dst_ref[0, 0, :] = dst0.astype(jnp.int32)
    dst_ref[0, 1, :] = dst1.astype(jnp.int32)
    run_ref[0, :] = (run + jnp.sum(occ, axis=0)).astype(jnp.int32)


def _dispatch(idx, map_out):
    return pl.pallas_call(
        _dispatch_body,
        grid=(_RG,),
        in_specs=[
            pl.BlockSpec((1, _K, _RB), lambda g: (g, 0, 0)),
            pl.BlockSpec((8, _E), lambda g: (0, 0)),
        ],
        out_specs=pl.BlockSpec((1, _K, _RB), lambda g: (g, 0, 0)),
        out_shape=jax.ShapeDtypeStruct((_RG, _K, _RB), jnp.int32),
        scratch_shapes=[pltpu.VMEM((8, _E), jnp.int32)],
        compiler_params=pltpu.CompilerParams(
            dimension_semantics=("arbitrary",)),
    )(idx, map_out)


def _ffn_body(meta_ref, xpad_ref, gu_ref, dp_ref, ypad_ref):
    g = pl.program_id(0)

    @pl.when(g < meta_ref[1, 0])
    def _compute():
        x = xpad_ref[...].astype(jnp.float32)
        y = jax.lax.dot_general(x, gu_ref[0], (((1,), (1,)), ((), ())),
                                preferred_element_type=jnp.float32)
        gate = y[:, :_F]
        up = y[:, _F:]
        h = gate * jax.nn.sigmoid(gate) * up
        o = jax.lax.dot_general(h, dp_ref[0], (((1,), (1,)), ((), ())),
                                preferred_element_type=jnp.float32)
        ypad_ref[...] = o.astype(jnp.bfloat16)


def _grouped_ffn(meta, xpad, gate_up_proj, down_proj):
    grid_spec = pltpu.PrefetchScalarGridSpec(
        num_scalar_prefetch=1,
        grid=(_G,),
        in_specs=[
            pl.BlockSpec((_TM, _D),
                         lambda g, m: (jnp.minimum(g, m[1, 0] - 1), 0)),
            pl.BlockSpec((1, 2 * _F, _D), lambda g, m: (m[0, g], 0, 0)),
            pl.BlockSpec((1, _D, _F), lambda g, m: (m[0, g], 0, 0)),
        ],
        out_specs=pl.BlockSpec((_TM, _D),
                               lambda g, m: (jnp.minimum(g, m[1, 0] - 1), 0)),
    )
    return pl.pallas_call(
        _ffn_body,
        grid_spec=grid_spec,
        out_shape=jax.ShapeDtypeStruct((_P, _D), jnp.bfloat16),
        compiler_params=pltpu.CompilerParams(
            dimension_semantics=("arbitrary",)),
    )(meta, xpad, gate_up_proj, down_proj)


def kernel(hidden_states, gate_weight, gate_up_proj, down_proj, similarity_matrix):
    bsz, seq, dim = hidden_states.shape
    tokens = bsz * seq
    x = hidden_states.reshape(tokens, dim)

    vals, idx, map_out, meta, xb = _router(x, gate_weight, similarity_matrix)
    top_vals = jnp.moveaxis(vals, 1, 2).reshape(tokens, _K)

    dst_out = _dispatch(idx, map_out)
    dst = jnp.moveaxis(dst_out, 1, 2).reshape(_K * tokens)

    pair_tok = (jnp.arange(_K * tokens, dtype=jnp.int32) // _K)
    tok_pad = jnp.zeros((_P,), jnp.int32).at[dst].set(pair_tok)

    xpad = jnp.take(xb, tok_pad, axis=0)

    ypad = _grouped_ffn(meta, xpad, gate_up_proj, down_proj)

    inv2 = dst.reshape(tokens, _K)
    out = (jnp.take(ypad, inv2[:, 0], axis=0).astype(jnp.float32)
           * top_vals[:, :1]
           + jnp.take(ypad, inv2[:, 1], axis=0).astype(jnp.float32)
           * top_vals[:, 1:])
    return out.reshape(bsz, seq, dim)


# speedup vs baseline: 1.0272x; 1.0272x over previous
import jax
import jax.numpy as jnp
from jax.experimental import pallas as pl
from jax.experimental.pallas import tpu as pltpu

_E = 64
_K = 2
_D = 1024
_F = 768
_TM = 128
_G = 95
_P = _G * _TM


_RB = 128
_RG = 16


def _router_body(x_ref, gw_ref, sim_ref, vals_ref, idx_ref, map_ref, meta_ref,
                 xb_ref):
    g = pl.program_id(0)
    xb_ref[...] = x_ref[...].astype(jnp.bfloat16)
    iota_e = jax.lax.broadcasted_iota(jnp.int32, (_RB, _E), 1)
    logits = jax.lax.dot_general(x_ref[...], gw_ref[...],
                                 (((1,), (1,)), ((), ())),
                                 preferred_element_type=jnp.float32)
    m = jnp.max(logits, axis=1, keepdims=True)
    e = jnp.exp(logits - m)
    p = e / jnp.sum(e, axis=1, keepdims=True)
    v0 = jnp.max(p, axis=1)
    i0 = jnp.min(jnp.where(p == v0[:, None], iota_e, _E), axis=1)
    p1 = jnp.where(iota_e == i0[:, None], -1.0, p)
    v1 = jnp.max(p1, axis=1)
    i1 = jnp.min(jnp.where(p1 == v1[:, None], iota_e, _E), axis=1)
    s = jnp.maximum(v0 + v1, 1e-12)
    vals_ref[0, 0, :] = v0 / s
    vals_ref[0, 1, :] = v1 / s
    idx_ref[0, 0, :] = i0
    idx_ref[0, 1, :] = i1

    pm_c = jnp.sum((iota_e == i0[:, None]).astype(jnp.int32), axis=0)
    sm_c = jnp.sum((iota_e == i1[:, None]).astype(jnp.int32), axis=0)

    @pl.when(g == 0)
    def _init():
        map_ref[...] = jnp.zeros_like(map_ref)

    map_ref[1, :] = map_ref[1, :] + pm_c
    map_ref[2, :] = map_ref[2, :] + sm_c

    @pl.when(g == _RG - 1)
    def _finish():
        counts0 = map_ref[1, :]
        counts1 = map_ref[2, :]
        pm = counts0 > 0
        sm = counts1 > 0
        sim = sim_ref[...]
        neg = jnp.float32(-3.0e38)
        cand = jnp.where(pm[None, :], sim, neg)
        best_sim = jnp.max(cand, axis=1)
        iota2 = jax.lax.broadcasted_iota(jnp.int32, (_E, _E), 1)
        best_primary = jnp.min(
            jnp.where(cand == best_sim[:, None], iota2, _E), axis=1)
        any_pm = jnp.sum(pm.astype(jnp.int32)) > 0
        any_sm = jnp.sum(sm.astype(jnp.int32)) > 0
        reroute = sm & (~pm) & any_pm & any_sm & (best_sim >= 0.5)
        ident = jax.lax.broadcasted_iota(jnp.int32, (_E,), 0)
        mapping = jnp.where(reroute, best_primary, ident)
        map_ref[0, :] = mapping

        oh_map = (iota2 == mapping[:, None]).astype(jnp.int32)
        counts_r = counts0 + jnp.sum(oh_map * counts1[:, None], axis=0)
        tile_counts = (counts_r + _TM - 1) // _TM
        r2 = jax.lax.broadcasted_iota(jnp.int32, (_E, _E), 0)
        tile_ends = jnp.sum(
            jnp.where(iota2 <= r2, tile_counts[None, :], 0), axis=1)
        map_ref[3, :] = (tile_ends - tile_counts) * _TM

        nt = jnp.max(tile_ends)
        te_b = jnp.broadcast_to(tile_ends[None, :], (128, _E))
        g_col = jax.lax.broadcasted_iota(jnp.int32, (128, _E), 0)
        gg_col = jnp.minimum(g_col, nt - 1)
        be = jnp.sum((te_b <= gg_col).astype(jnp.int32), axis=1)
        meta_ref[0, :] = be
        meta_ref[1, :] = jnp.full((128,), nt, jnp.int32)


def _router(x, gate_weight, similarity_matrix):
    return pl.pallas_call(
        _router_body,
        grid=(_RG,),
        in_specs=[
            pl.BlockSpec((_RB, _D), lambda g: (g, 0)),
            pl.BlockSpec((_E, _D), lambda g: (0, 0)),
            pl.BlockSpec((_E, _E), lambda g: (0, 0)),
        ],
        out_specs=[
            pl.BlockSpec((1, _K, _RB), lambda g: (g, 0, 0)),
            pl.BlockSpec((1, _K, _RB), lambda g: (g, 0, 0)),
            pl.BlockSpec((8, _E), lambda g: (0, 0)),
            pl.BlockSpec((8, 128), lambda g: (0, 0)),
            pl.BlockSpec((_RB, _D), lambda g: (g, 0)),
        ],
        out_shape=[
            jax.ShapeDtypeStruct((_RG, _K, _RB), jnp.float32),
            jax.ShapeDtypeStruct((_RG, _K, _RB), jnp.int32),
            jax.ShapeDtypeStruct((8, _E), jnp.int32),
            jax.ShapeDtypeStruct((8, 128), jnp.int32),
            jax.ShapeDtypeStruct((_RG * _RB, _D), jnp.bfloat16),
        ],
        compiler_params=pltpu.CompilerParams(
            dimension_semantics=("arbitrary",)),
    )(x, gate_weight, similarity_matrix)


def _dispatch_body(idx_ref, map_ref, dst_ref, run_ref):
    g = pl.program_id(0)
    iota_e = jax.lax.broadcasted_iota(jnp.int32, (_RB, _E), 1)
    i0 = idx_ref[0, 0, :]
    i1 = idx_ref[0, 1, :]
    mapping = map_ref[0, :].astype(jnp.float32)
    pad_off = map_ref[3, :].astype(jnp.float32)

    oh1 = (iota_e == i1[:, None]).astype(jnp.float32)
    e1 = jnp.sum(oh1 * mapping[None, :], axis=1).astype(jnp.int32)
    oh0 = (iota_e == i0[:, None]).astype(jnp.float32)
    oh1m = (iota_e == e1[:, None]).astype(jnp.float32)
    occ = oh0 + oh1m

    r = jax.lax.broadcasted_iota(jnp.int32, (_RB, _RB), 0)
    c = jax.lax.broadcasted_iota(jnp.int32, (_RB, _RB), 1)
    lstrict = (r > c).astype(jnp.float32)
    cnt_lt = jax.lax.dot_general(lstrict, occ, (((1,), (0,)), ((), ())),
                                 preferred_element_type=jnp.float32)

    @pl.when(g == 0)
    def _init():
        run_ref[...] = jnp.zeros_like(run_ref)

    run = run_ref[0, :].astype(jnp.float32)
    base = run[None, :] + cnt_lt + pad_off[None, :]
    dst0 = jnp.sum(oh0 * base, axis=1)
    dst1 = jnp.sum(oh1m * (base + oh0), axis=1)
    dst_ref[0, 0, :] = dst0.astype(jnp.int32)
    dst_ref[0, 1, :] = dst1.astype(jnp.int32)
    run_ref[0, :] = (run + jnp.sum(occ, axis=0)).astype(jnp.int32)


def _dispatch(idx, map_out):
    return pl.pallas_call(
        _dispatch_body,
        grid=(_RG,),
        in_specs=[
            pl.BlockSpec((1, _K, _RB), lambda g: (g, 0, 0)),
            pl.BlockSpec((8, _E), lambda g: (0, 0)),
        ],
        out_specs=pl.BlockSpec((1, _K, _RB), lambda g: (g, 0, 0)),
        out_shape=jax.ShapeDtypeStruct((_RG, _K, _RB), jnp.int32),
        scratch_shapes=[pltpu.VMEM((8, _E), jnp.int32)],
        compiler_params=pltpu.CompilerParams(
            dimension_semantics=("arbitrary",)),
    )(idx, map_out)


def _ffn_body(meta_ref, xpad_ref, gu_ref, dp_ref, ypad_ref):
    g = pl.program_id(0)

    @pl.when(g < meta_ref[1, 0])
    def _compute():
        x = xpad_ref[...].astype(jnp.float32)
        y = jax.lax.dot_general(x, gu_ref[0], (((1,), (1,)), ((), ())),
                                preferred_element_type=jnp.float32)
        gate = y[:, :_F]
        up = y[:, _F:]
        h = gate * jax.nn.sigmoid(gate) * up
        o = jax.lax.dot_general(h, dp_ref[0], (((1,), (1,)), ((), ())),
                                preferred_element_type=jnp.float32)
        ypad_ref[...] = o.astype(jnp.bfloat16)


def _grouped_ffn(meta, xpad, gate_up_proj, down_proj):
    grid_spec = pltpu.PrefetchScalarGridSpec(
        num_scalar_prefetch=1,
        grid=(_G,),
        in_specs=[
            pl.BlockSpec((_TM, _D),
                         lambda g, m: (jnp.minimum(g, m[1, 0] - 1), 0)),
            pl.BlockSpec((1, 2 * _F, _D), lambda g, m: (m[0, g], 0, 0)),
            pl.BlockSpec((1, _D, _F), lambda g, m: (m[0, g], 0, 0)),
        ],
        out_specs=pl.BlockSpec((_TM, _D),
                               lambda g, m: (jnp.minimum(g, m[1, 0] - 1), 0)),
    )
    return pl.pallas_call(
        _ffn_body,
        grid_spec=grid_spec,
        out_shape=jax.ShapeDtypeStruct((_P, _D), jnp.bfloat16),
        compiler_params=pltpu.CompilerParams(
            dimension_semantics=("arbitrary",)),
    )(meta, xpad, gate_up_proj, down_proj)


def kernel(hidden_states, gate_weight, gate_up_proj, down_proj, similarity_matrix):
    bsz, seq, dim = hidden_states.shape
    tokens = bsz * seq
    x = hidden_states.reshape(tokens, dim)

    vals, idx, map_out, meta, xb = _router(x, gate_weight, similarity_matrix)
    top_vals = jnp.moveaxis(vals, 1, 2).reshape(tokens, _K)

    dst_out = _dispatch(idx, map_out)
    dst = jnp.moveaxis(dst_out, 1, 2).reshape(_K * tokens)

    pair_tok = (jnp.arange(_K * tokens, dtype=jnp.int32) // _K)
    tok_pad = jnp.zeros((_P,), jnp.int32).at[dst].set(pair_tok)

    xpad = jnp.take(xb, tok_pad, axis=0)

    ypad = _grouped_ffn(meta, xpad, gate_up_proj, down_proj)

    inv_all = jnp.transpose(dst_out, (1, 0, 2)).reshape(_K * tokens)
    yg = jnp.take(ypad, inv_all, axis=0)
    out = (yg[:tokens].astype(jnp.float32) * top_vals[:, :1]
           + yg[tokens:].astype(jnp.float32) * top_vals[:, 1:])
    return out.reshape(bsz, seq, dim)
